# baseline (speedup 1.0000x reference)
import jax
import jax.numpy as jnp
from jax.experimental import pallas as pl
from jax.experimental.pallas import tpu as pltpu

_BATCH = 16384
_IN = 256
_HID = 256
_NSLOT = 24
_K = 6
_TILE = 4096

_SENTINEL = -2147483648


def _body(x_ref, w1_ref, b1_ref, w2_ref, b2_ref, w3_ref, b3_ref, o_ref):
    x = x_ref[...]
    h = jnp.dot(x, w1_ref[...], preferred_element_type=jnp.float32) + b1_ref[...]
    h = jnp.maximum(h, 0.0)
    h = jnp.dot(h, w2_ref[...], preferred_element_type=jnp.float32) + b2_ref[...]
    h = jnp.maximum(h, 0.0)
    lt = jax.lax.dot_general(
        w3_ref[...], h, (((1,), (1,)), ((), ())),
        preferred_element_type=jnp.float32) + b3_ref[...]
    row = jax.lax.broadcasted_iota(jnp.int32, lt.shape, 0)
    selectable = (row != 0) & (row != 12)
    u = lt.view(jnp.int32)
    k = u ^ ((u >> 31) & jnp.int32(0x7FFFFFFF))
    k = (k & jnp.int32(-32)) | (jnp.int32(31) - row)
    work = jnp.where(selectable, k, jnp.int32(_SENTINEL))
    acc = jnp.where(selectable, 0.0, 1.0)
    for _ in range(_K):
        m = jnp.max(work, axis=0, keepdims=True)
        pick = work == m
        acc = jnp.where(pick, 1.0, acc)
        work = jnp.where(pick, jnp.int32(_SENTINEL), work)
    o_ref[...] = acc


@jax.jit
def kernel(qoi_features, W1, b1, W2, b2, W3, b3):
    out = pl.pallas_call(
        _body,
        grid=(_BATCH // _TILE,),
        in_specs=[
            pl.BlockSpec((_TILE, _IN), lambda i: (i, 0)),
            pl.BlockSpec((_IN, _HID), lambda i: (0, 0)),
            pl.BlockSpec((1, _HID), lambda i: (0, 0)),
            pl.BlockSpec((_HID, _HID), lambda i: (0, 0)),
            pl.BlockSpec((1, _HID), lambda i: (0, 0)),
            pl.BlockSpec((_NSLOT, _HID), lambda i: (0, 0)),
            pl.BlockSpec((_NSLOT, 1), lambda i: (0, 0)),
        ],
        out_specs=pl.BlockSpec((_NSLOT, _TILE), lambda i: (0, i)),
        out_shape=jax.ShapeDtypeStruct((_NSLOT, _BATCH), jnp.float32),
        compiler_params=pltpu.CompilerParams(
            dimension_semantics=("arbitrary",),
            vmem_limit_bytes=100 * 1024 * 1024,
        ),
    )(qoi_features, W1, b1.reshape(1, _HID), W2, b2.reshape(1, _HID),
      W3.T, b3.reshape(_NSLOT, 1))
    return out.reshape(2, 12, _BATCH).transpose(2, 0, 1)


# speedup vs baseline: 1.0044x; 1.0044x over previous
import jax
import jax.numpy as jnp
from jax.experimental import pallas as pl

_BATCH = 16384
_IN = 256
_HID = 256
_NSLOT = 24
_K = 6
_TILE = 4096

_SENTINEL = -2147483648


def _body(x_ref, w1_ref, b1_ref, w2_ref, b2_ref, w3_ref, b3_ref, o_ref):
    x = x_ref[...]
    h = jnp.dot(x, w1_ref[...], preferred_element_type=jnp.float32) + b1_ref[...]
    h = jnp.maximum(h, 0.0)
    h = jnp.dot(h, w2_ref[...], preferred_element_type=jnp.float32) + b2_ref[...]
    h = jnp.maximum(h, 0.0)
    lt = jax.lax.dot_general(
        w3_ref[...], h, (((1,), (1,)), ((), ())),
        preferred_element_type=jnp.float32) + b3_ref[...]
    row = jax.lax.broadcasted_iota(jnp.int32, lt.shape, 0)
    selectable = (row != 0) & (row != 12)
    u = lt.view(jnp.int32)
    k = u ^ ((u >> 31) & jnp.int32(0x7FFFFFFF))
    k = (k & jnp.int32(-32)) | (jnp.int32(31) - row)
    work = jnp.where(selectable, k, jnp.int32(_SENTINEL))
    acc = jnp.where(selectable, 0.0, 1.0)
    for _ in range(_K):
        m = jnp.max(work, axis=0, keepdims=True)
        pick = work == m
        acc = jnp.where(pick, 1.0, acc)
        work = jnp.where(pick, jnp.int32(_SENTINEL), work)
    o_ref[...] = acc


@jax.jit
def kernel(qoi_features, W1, b1, W2, b2, W3, b3):
    out = pl.pallas_call(
        _body,
        grid=(_BATCH // _TILE,),
        in_specs=[
            pl.BlockSpec((_TILE, _IN), lambda i: (i, 0)),
            pl.BlockSpec((_IN, _HID), lambda i: (0, 0)),
            pl.BlockSpec((1, _HID), lambda i: (0, 0)),
            pl.BlockSpec((_HID, _HID), lambda i: (0, 0)),
            pl.BlockSpec((1, _HID), lambda i: (0, 0)),
            pl.BlockSpec((_NSLOT, _HID), lambda i: (0, 0)),
            pl.BlockSpec((_NSLOT, 1), lambda i: (0, 0)),
        ],
        out_specs=pl.BlockSpec((_NSLOT, _TILE), lambda i: (0, i)),
        out_shape=jax.ShapeDtypeStruct((_NSLOT, _BATCH), jnp.float32),
    )(qoi_features, W1, b1.reshape(1, _HID), W2, b2.reshape(1, _HID),
      W3.T, b3.reshape(_NSLOT, 1))
    return out.T.reshape(_BATCH, 2, 12)
